# baseline (speedup 1.0000x reference)
import functools

import jax
import jax.numpy as jnp
from jax.experimental import pallas as pl
from jax.experimental.pallas import tpu as pltpu
from jax.experimental.pallas import tpu_sc as plsc


def kernel(inputs, word_table, pos_table):
    B, L = inputs.shape
    V, E = word_table.shape
    N = B * L

    idx_flat = inputs.reshape(B, 1, L).astype(jnp.int32)
    pe = pos_table[:L]

    mesh = plsc.VectorSubcoreMesh(core_axis_name="c", subcore_axis_name="s")

    @functools.partial(
        pl.kernel,
        out_type=jax.ShapeDtypeStruct((N, E), jnp.float32),
        mesh=mesh,
        scratch_types=[pltpu.VMEM((L, E), jnp.float32)],
        compiler_params=pltpu.CompilerParams(use_tc_tiling_on_sc=False),
    )
    def emb(w_hbm, i_hbm, p_hbm, o_hbm, pe_v):
        pltpu.sync_copy(p_hbm, pe_v)

        def body(i_vmem, o_vmem):
            pltpu.sync_copy(w_hbm.at[i_vmem.at[0, 0]], o_vmem)

            @pl.loop(0, L)
            def _(r):
                for c in range(0, E, 16):
                    slc = (pl.ds(r, 1), pl.ds(c, 16))
                    o_vmem.at[*slc][...] = o_vmem.at[*slc][...] + pe_v.at[*slc][...]

        pltpu.emit_pipeline(
            body,
            grid=(B,),
            in_specs=[pl.BlockSpec((1, 1, L), lambda i: (i, 0, 0))],
            out_specs=[pl.BlockSpec((L, E), lambda i: (i, 0))],
            core_axis_name=("c", "s"),
            dimension_semantics=(pltpu.PARALLEL,),
        )(i_hbm, o_hbm)

    out = emb(word_table, idx_flat, pe)
    return out.reshape(B, L, E)
